# Initial kernel scaffold; baseline (speedup 1.0000x reference)
import jax
import jax.numpy as jnp
from jax.experimental import pallas as pl


def kernel(points_xyz, features, params):
    raise NotImplementedError("write your pallas kernel here")


# speedup vs baseline: 1.3871x; 1.3871x over previous
import functools

import jax
import jax.numpy as jnp
import numpy as np
from jax import lax
from jax.experimental import pallas as pl
from jax.experimental.pallas import tpu as pltpu

_NUM_POINT = 1024
_RADII = (0.2, 0.4)
_SAMPLE_NUMS = (16, 32)
_BN_EPS = 1e-5


def _fps_body(xyz_ref, idx_ref, nxyz_ref, *, n, npoint):
    rows = n // 128
    X = xyz_ref[0, 0]
    Y = xyz_ref[0, 1]
    Z = xyz_ref[0, 2]
    lin = (lax.broadcasted_iota(jnp.int32, (rows, 128), 0) * 128
           + lax.broadcasted_iota(jnp.int32, (rows, 128), 1))
    prow = npoint // 128
    pos = (lax.broadcasted_iota(jnp.int32, (prow, 128), 0) * 128
           + lax.broadcasted_iota(jnp.int32, (prow, 128), 1))

    def body(i, state):
        dists, far, iacc, xacc, yacc, zacc = state
        sel = lin == far
        cx = jnp.sum(jnp.where(sel, X, 0.0))
        cy = jnp.sum(jnp.where(sel, Y, 0.0))
        cz = jnp.sum(jnp.where(sel, Z, 0.0))
        iacc = jnp.where(pos == i, far, iacc)
        xacc = jnp.where(pos == i, cx, xacc)
        yacc = jnp.where(pos == i, cy, yacc)
        zacc = jnp.where(pos == i, cz, zacc)
        dx = X - cx
        dy = Y - cy
        dz = Z - cz
        d = dx * dx + dy * dy + dz * dz
        dists = jnp.minimum(dists, d)
        m = jnp.max(dists)
        far2 = jnp.min(jnp.where(dists == m, lin, n)).astype(jnp.int32)
        return (dists, far2, iacc, xacc, yacc, zacc)

    dists0 = jnp.full((rows, 128), 1e10, jnp.float32)
    far0 = jnp.int32(0)
    iacc0 = jnp.zeros((prow, 128), jnp.int32)
    zf = jnp.zeros((prow, 128), jnp.float32)
    _, _, iacc, xacc, yacc, zacc = lax.fori_loop(
        0, npoint, body, (dists0, far0, iacc0, zf, zf, zf))
    idx_ref[0] = iacc
    nxyz_ref[0, 0] = xacc
    nxyz_ref[0, 1] = yacc
    nxyz_ref[0, 2] = zacc


def _fps(points_xyz, npoint, interpret=False):
    B, N, _ = points_xyz.shape
    xyz_t = jnp.transpose(points_xyz, (0, 2, 1)).reshape(B, 3, N // 128, 128)
    prow = npoint // 128
    idx, nxyz = pl.pallas_call(
        functools.partial(_fps_body, n=N, npoint=npoint),
        grid=(B,),
        in_specs=[pl.BlockSpec((1, 3, N // 128, 128), lambda b: (b, 0, 0, 0))],
        out_specs=[pl.BlockSpec((1, prow, 128), lambda b: (b, 0, 0)),
                   pl.BlockSpec((1, 3, prow, 128), lambda b: (b, 0, 0, 0))],
        out_shape=[jax.ShapeDtypeStruct((B, prow, 128), jnp.int32),
                   jax.ShapeDtypeStruct((B, 3, prow, 128), jnp.float32)],
        interpret=interpret,
    )(xyz_t)
    indices = idx.reshape(B, npoint)
    new_xyz = jnp.transpose(nxyz.reshape(B, 3, npoint), (0, 2, 1))
    return indices, new_xyz


def _ball_query_jnp(sqrdists, radius, nsample):
    B, S, N = sqrdists.shape
    mask = sqrdists <= radius ** 2
    arange = jnp.arange(N, dtype=jnp.int32)[None, None, :]
    gidx = jnp.where(mask, arange, N)
    gidx = jnp.sort(gidx, axis=-1)[:, :, :nsample]
    first = gidx[:, :, :1]
    gidx = jnp.where(gidx == N, jnp.broadcast_to(first, gidx.shape), gidx)
    return gidx


def _gather3(points, idx):
    B = points.shape[0]
    return points[jnp.arange(B)[:, None, None], idx]


def _mlp_jnp(x, layers):
    for layer in layers:
        x = jnp.einsum('oc,bcsk->bosk', layer['w'], x) + layer['b'][None, :, None, None]
        x = (x / jnp.sqrt(1.0 + _BN_EPS)) * layer['gamma'][None, :, None, None] + layer['beta'][None, :, None, None]
        x = jax.nn.relu(x)
    return x


def kernel(points_xyz, features, params):
    B, N, _ = points_xyz.shape
    indices, new_xyz = _fps(points_xyz, _NUM_POINT)
    sqrdists = jnp.sum((new_xyz[:, :, None, :] - points_xyz[:, None, :, :]) ** 2, axis=-1)
    feats_t = jnp.transpose(features, (0, 2, 1))
    outs = []
    for i in range(len(_RADII)):
        gidx = _ball_query_jnp(sqrdists, _RADII[i], _SAMPLE_NUMS[i])
        grouped_xyz = _gather3(points_xyz, gidx) - new_xyz[:, :, None, :]
        grouped_feats = _gather3(feats_t, gidx)
        grouped = jnp.concatenate([grouped_xyz, grouped_feats], axis=-1)
        x = jnp.transpose(grouped, (0, 3, 1, 2))
        x = _mlp_jnp(x, params['mlp%d' % i])
        outs.append(jnp.max(x, axis=-1))
    new_features = jnp.concatenate(outs, axis=1)
    return (new_xyz, new_features, indices)


# speedup vs baseline: 15.0424x; 10.8443x over previous
import functools

import jax
import jax.numpy as jnp
import numpy as np
from jax import lax
from jax.experimental import pallas as pl
from jax.experimental.pallas import tpu as pltpu
from jax.experimental.pallas import tpu_sc as plsc

_NUM_POINT = 1024
_RADII = (0.2, 0.4)
_SAMPLE_NUMS = (16, 32)
_BN_EPS = 1e-5


def _fps_body(xyz_ref, idx_ref, nxyz_ref, *, n, npoint):
    rows = n // 128
    X = xyz_ref[0, 0]
    Y = xyz_ref[0, 1]
    Z = xyz_ref[0, 2]
    lin = (lax.broadcasted_iota(jnp.int32, (rows, 128), 0) * 128
           + lax.broadcasted_iota(jnp.int32, (rows, 128), 1))
    prow = npoint // 128
    pos = (lax.broadcasted_iota(jnp.int32, (prow, 128), 0) * 128
           + lax.broadcasted_iota(jnp.int32, (prow, 128), 1))

    def body(i, state):
        dists, far, iacc, xacc, yacc, zacc = state
        sel = lin == far
        cx = jnp.sum(jnp.where(sel, X, 0.0))
        cy = jnp.sum(jnp.where(sel, Y, 0.0))
        cz = jnp.sum(jnp.where(sel, Z, 0.0))
        iacc = jnp.where(pos == i, far, iacc)
        xacc = jnp.where(pos == i, cx, xacc)
        yacc = jnp.where(pos == i, cy, yacc)
        zacc = jnp.where(pos == i, cz, zacc)
        dx = X - cx
        dy = Y - cy
        dz = Z - cz
        d = dx * dx + dy * dy + dz * dz
        dists = jnp.minimum(dists, d)
        m = jnp.max(dists)
        far2 = jnp.min(jnp.where(dists == m, lin, n)).astype(jnp.int32)
        return (dists, far2, iacc, xacc, yacc, zacc)

    dists0 = jnp.full((rows, 128), 1e10, jnp.float32)
    far0 = jnp.int32(0)
    iacc0 = jnp.zeros((prow, 128), jnp.int32)
    zf = jnp.zeros((prow, 128), jnp.float32)
    _, _, iacc, xacc, yacc, zacc = lax.fori_loop(
        0, npoint, body, (dists0, far0, iacc0, zf, zf, zf))
    idx_ref[0] = iacc
    nxyz_ref[0, 0] = xacc
    nxyz_ref[0, 1] = yacc
    nxyz_ref[0, 2] = zacc


def _fps(points_xyz, npoint, interpret=False):
    B, N, _ = points_xyz.shape
    xyz_t = jnp.transpose(points_xyz, (0, 2, 1)).reshape(B, 3, N // 128, 128)
    prow = npoint // 128
    idx, nxyz = pl.pallas_call(
        functools.partial(_fps_body, n=N, npoint=npoint),
        grid=(B,),
        in_specs=[pl.BlockSpec((1, 3, N // 128, 128), lambda b: (b, 0, 0, 0))],
        out_specs=[pl.BlockSpec((1, prow, 128), lambda b: (b, 0, 0)),
                   pl.BlockSpec((1, 3, prow, 128), lambda b: (b, 0, 0, 0))],
        out_shape=[jax.ShapeDtypeStruct((B, prow, 128), jnp.int32),
                   jax.ShapeDtypeStruct((B, 3, prow, 128), jnp.float32)],
        interpret=interpret,
    )(xyz_t)
    indices = idx.reshape(B, npoint)
    nxyz_t = nxyz.reshape(B, 3, npoint)
    new_xyz = jnp.transpose(nxyz_t, (0, 2, 1))
    return indices, new_xyz, nxyz_t


def _aprep_body(ft_ref, xyz_ref, wf0_ref, wx0_ref, b0_ref, wf1_ref, wx1_ref,
                b1_ref, a0_ref):
    f = ft_ref[0]
    p = xyz_ref[0]
    px0 = jax.lax.dot_general(p, wx0_ref[...], (((1,), (0,)), ((), ())),
                              preferred_element_type=jnp.float32)
    px1 = jax.lax.dot_general(p, wx1_ref[...], (((1,), (0,)), ((), ())),
                              preferred_element_type=jnp.float32)
    a0 = jax.lax.dot_general(f, wf0_ref[...], (((1,), (0,)), ((), ())),
                             preferred_element_type=jnp.float32) + px0 + b0_ref[...]
    a1 = jax.lax.dot_general(f, wf1_ref[...], (((1,), (0,)), ((), ())),
                             preferred_element_type=jnp.float32) + px1 + b1_ref[...]
    a0_ref[0] = jnp.concatenate([a0, a1], axis=-1)


def _aprep(feats_t, points_xyz, fold0, fold1, interpret=False):
    B, N, _ = feats_t.shape
    blk = 2048
    full = lambda s: pl.BlockSpec(s, lambda b, n: (0,) * len(s))
    a = pl.pallas_call(
        _aprep_body,
        grid=(B, N // blk),
        in_specs=[pl.BlockSpec((1, blk, 64), lambda b, n: (b, n, 0)),
                  pl.BlockSpec((1, blk, 3), lambda b, n: (b, n, 0)),
                  full((64, 64)), full((3, 64)), full((1, 64)),
                  full((64, 64)), full((3, 64)), full((1, 64))],
        out_specs=pl.BlockSpec((1, blk, 128), lambda b, n: (b, n, 0)),
        out_shape=jax.ShapeDtypeStruct((B, N, 128), jnp.float32),
        interpret=interpret,
    )(feats_t, points_xyz, fold0['wf'], fold0['wx'], fold0['b'],
      fold1['wf'], fold1['wx'], fold1['b'])
    return a


def _fold_params(layers):
    out = []
    for layer in layers:
        t = layer['gamma'] / jnp.sqrt(1.0 + _BN_EPS)
        wt = layer['w'].T * t[None, :]
        bt = layer['b'] * t + layer['beta']
        out.append((wt, bt))
    return out


def _mlp_body(g_ref, nxyz_ref, wx_ref, w2_ref, b2_ref, w3_ref, b3_ref,
              out_ref, *, nsample, sb, half):
    nx = nxyz_ref[0]
    c = -jax.lax.dot_general(nx, wx_ref[...], (((0,), (0,)), ((), ())),
                             preferred_element_type=jnp.float32)
    g = g_ref[0][:, half * 64:half * 64 + 64]
    x1 = jax.nn.relu(g.reshape(sb, nsample, 64) + c[:, None, :])
    x1 = x1.reshape(sb * nsample, 64)
    x2 = jax.nn.relu(
        jax.lax.dot_general(x1, w2_ref[...], (((1,), (0,)), ((), ())),
                            preferred_element_type=jnp.float32) + b2_ref[...])
    x3 = jax.nn.relu(
        jax.lax.dot_general(x2, w3_ref[...], (((1,), (0,)), ((), ())),
                            preferred_element_type=jnp.float32) + b3_ref[...])
    out_ref[0] = jnp.max(x3.reshape(sb, nsample, 128), axis=1)


def _mlp(g, nxyz_t, wx, w2, b2, w3, b3, nsample, half, interpret=False):
    B = nxyz_t.shape[0]
    S = nxyz_t.shape[2]
    sb = 128
    h2 = w2.shape[1]
    full = lambda s: pl.BlockSpec(s, lambda b, n: (0,) * len(s))
    out = pl.pallas_call(
        functools.partial(_mlp_body, nsample=nsample, sb=sb, half=half),
        grid=(B, S // sb),
        in_specs=[pl.BlockSpec((1, sb * nsample, 128), lambda b, n: (b, n, 0)),
                  pl.BlockSpec((1, 3, sb), lambda b, n: (b, 0, n)),
                  full((3, 64)), full((64, h2)), full((1, h2)),
                  full((h2, 128)), full((1, 128))],
        out_specs=pl.BlockSpec((1, sb, 128), lambda b, n: (b, n, 0)),
        out_shape=jax.ShapeDtypeStruct((B, S, 128), jnp.float32),
        interpret=interpret,
    )(g.reshape(B, S * nsample, 128), nxyz_t, wx, w2, b2, w3, b3)
    return out


def _ballquery_gather_sc(xyz_t, nxyz_t, av, npoint):
    B, _, N = xyz_t.shape
    S = npoint
    NW = 32
    CPW = (B * S) // NW
    K0, K1 = _SAMPLE_NUMS
    r0sq = jnp.float32(float(_RADII[0]) ** 2)
    r1sq = jnp.float32(float(_RADII[1]) ** 2)
    nsteps = N // 16
    wpb = NW // B
    mesh = plsc.VectorSubcoreMesh(core_axis_name="c", subcore_axis_name="s")

    @functools.partial(
        pl.kernel, mesh=mesh,
        compiler_params=pltpu.CompilerParams(needs_layout_passes=False),
        out_type=[jax.ShapeDtypeStruct((B * S * K0, 128), jnp.float32),
                  jax.ShapeDtypeStruct((B * S * K1, 128), jnp.float32)],
        scratch_types=[pltpu.VMEM((N,), jnp.float32),
                       pltpu.VMEM((N,), jnp.float32),
                       pltpu.VMEM((N,), jnp.float32),
                       pltpu.VMEM((CPW + 16,), jnp.float32),
                       pltpu.VMEM((CPW + 16,), jnp.float32),
                       pltpu.VMEM((CPW + 16,), jnp.float32),
                       pltpu.VMEM((K0 + 16,), jnp.int32),
                       pltpu.VMEM((K1 + 16,), jnp.int32),
                       pltpu.VMEM((CPW * K0 // 128, 128), jnp.int32),
                       pltpu.VMEM((CPW * K1 // 128, 128), jnp.int32),
                       pltpu.VMEM((256, 128), jnp.float32),
                       pltpu.VMEM((256, 128), jnp.float32),
                       pltpu.SemaphoreType.DMA,
                       pltpu.SemaphoreType.DMA,
                       pltpu.SemaphoreType.DMA],
    )
    def sc_kernel(xyz_hbm, nxyz_hbm, a_hbm, g0_hbm, g1_hbm,
                  xv, yv, zv, cxv, cyv, czv, s0, s1, i0, i1,
                  ring_a, ring_b, gsem, osem_a, osem_b):
        wid = lax.axis_index("s") * 2 + lax.axis_index("c")
        b = wid // wpb
        s_base = (wid % wpb) * CPW
        pltpu.sync_copy(xyz_hbm.at[pl.ds((b * 3 + 0) * N, N)], xv)
        pltpu.sync_copy(xyz_hbm.at[pl.ds((b * 3 + 1) * N, N)], yv)
        pltpu.sync_copy(xyz_hbm.at[pl.ds((b * 3 + 2) * N, N)], zv)
        pltpu.sync_copy(nxyz_hbm.at[pl.ds((b * 3 + 0) * S + s_base, CPW)],
                        cxv.at[pl.ds(0, CPW)])
        pltpu.sync_copy(nxyz_hbm.at[pl.ds((b * 3 + 1) * S + s_base, CPW)],
                        cyv.at[pl.ds(0, CPW)])
        pltpu.sync_copy(nxyz_hbm.at[pl.ds((b * 3 + 2) * S + s_base, CPW)],
                        czv.at[pl.ds(0, CPW)])
        lane = lax.broadcasted_iota(jnp.int32, (16,), 0)
        row_off = b * N

        def center_body(ci, carry):
            cx = cxv[pl.ds(ci, 16)][0]
            cy = cyv[pl.ds(ci, 16)][0]
            cz = czv[pl.ds(ci, 16)][0]

            def cond(st):
                i, c0, c1 = st
                return jnp.logical_and(
                    i < nsteps, jnp.logical_or(c0 < K0, c1 < K1))

            def step(st):
                i, c0, c1 = st
                xs = xv[pl.ds(i * 16, 16)]
                ys = yv[pl.ds(i * 16, 16)]
                zs = zv[pl.ds(i * 16, 16)]
                dx = xs - cx
                dy = ys - cy
                dz = zs - cz
                d = dx * dx + dy * dy + dz * dz
                m0 = d <= r0sq
                m1 = d <= r1sq
                gi = lane + i * 16
                plsc.store_compressed(s0.at[pl.ds(c0, 16)], gi, mask=m0)
                plsc.store_compressed(s1.at[pl.ds(c1, 16)], gi, mask=m1)
                c0 = jnp.minimum(c0 + jnp.sum(m0.astype(jnp.int32)), K0)
                c1 = jnp.minimum(c1 + jnp.sum(m1.astype(jnp.int32)), K1)
                return (i + 1, c0, c1)

            _, c0f, c1f = lax.while_loop(
                cond, step, (jnp.int32(0), jnp.int32(0), jnp.int32(0)))

            v0 = s0[pl.ds(0, 16)] + row_off
            first0 = v0[0]
            i0[ci // 8, pl.ds((ci % 8) * 16, 16)] = jnp.where(
                lane < c0f, v0, first0)
            va = s1[pl.ds(0, 16)] + row_off
            vb = s1[pl.ds(16, 16)] + row_off
            first1 = va[0]
            r1 = ci // 4
            col1 = (ci % 4) * 32
            i1[r1, pl.ds(col1, 16)] = jnp.where(lane < c1f, va, first1)
            i1[r1, pl.ds(col1 + 16, 16)] = jnp.where(
                lane + 16 < c1f, vb, first1)
            return carry

        lax.fori_loop(0, CPW, center_body, jnp.int32(0))

        rings = (ring_a, ring_b)
        osems = (osem_a, osem_b)

        def gather_stage(g_hbm, ibuf, k):
            total = CPW * k
            n_outer = total // 256
            out_base = wid * total
            out_handles = [None, None]
            for og in range(n_outer):
                p = og % 2
                if out_handles[p] is not None:
                    out_handles[p].wait()
                    out_handles[p] = None
                hs = []
                for j in range(2):
                    hs.append(pltpu.async_copy(
                        a_hbm.at[ibuf.at[og * 2 + j]],
                        rings[p].at[pl.ds(j * 128, 128)], gsem))
                for h in hs:
                    h.wait()
                out_handles[p] = pltpu.async_copy(
                    rings[p], g_hbm.at[pl.ds(out_base + og * 256, 256)],
                    osems[p])
            for h in out_handles:
                if h is not None:
                    h.wait()

        gather_stage(g0_hbm, i0, K0)
        gather_stage(g1_hbm, i1, K1)

    return sc_kernel(xyz_t.reshape(-1), nxyz_t.reshape(-1), av)


def _ball_query_jnp(sqrdists, radius, nsample):
    B, S, N = sqrdists.shape
    mask = sqrdists <= radius ** 2
    arange = jnp.arange(N, dtype=jnp.int32)[None, None, :]
    gidx = jnp.where(mask, arange, N)
    gidx = jnp.sort(gidx, axis=-1)[:, :, :nsample]
    first = gidx[:, :, :1]
    gidx = jnp.where(gidx == N, jnp.broadcast_to(first, gidx.shape), gidx)
    return gidx


def _kernel_impl(points_xyz, features, params, interpret=False):
    B, N, _ = points_xyz.shape
    indices, new_xyz, nxyz_t = _fps(points_xyz, _NUM_POINT, interpret=interpret)
    feats_t = jnp.transpose(features, (0, 2, 1))
    folds = []
    for i in range(len(_RADII)):
        fl = _fold_params(params['mlp%d' % i])
        folds.append({'wx': fl[0][0][:3], 'wf': fl[0][0][3:], 'b': fl[0][1][None, :],
                      'w2': fl[1][0], 'b2': fl[1][1][None, :],
                      'w3': fl[2][0], 'b3': fl[2][1][None, :]})
    a = _aprep(feats_t, points_xyz, folds[0], folds[1], interpret=interpret)
    av = a.reshape(B * N, 128)

    xyz_t = jnp.transpose(points_xyz, (0, 2, 1))
    g0, g1 = _ballquery_gather_sc(xyz_t, nxyz_t, av, _NUM_POINT)
    outs = []
    for i, g in enumerate((g0, g1)):
        fd = folds[i]
        o = _mlp(g, nxyz_t, fd['wx'], fd['w2'], fd['b2'], fd['w3'], fd['b3'],
                 _SAMPLE_NUMS[i], i, interpret=interpret)
        outs.append(jnp.transpose(o, (0, 2, 1)))
    new_features = jnp.concatenate(outs, axis=1)
    return (new_xyz, new_features, indices)


def kernel(points_xyz, features, params):
    return _kernel_impl(points_xyz, features, params)
